# baseline (speedup 1.0000x reference)
import jax
import jax.numpy as jnp
from jax import lax
from jax.experimental import pallas as pl
from jax.experimental.pallas import tpu as pltpu

_CHUNK = 8


def _explicit_gemm(lhs_k, w_ref, ncols, mxus=(0, 1)):
    M = lhs_k[0].shape[0]
    cols = [None] * ncols
    per_mxu = {}
    for c in range(ncols):
        m = mxus[c % len(mxus)]
        per_mxu.setdefault(m, []).append(c)
    seqs = list(per_mxu.items())
    depth = max(len(v) for _, v in seqs)
    for i in range(depth):
        for m, clist in seqs:
            if i < len(clist):
                c = clist[i]
                for k in range(2):
                    pltpu.matmul_push_rhs(w_ref[k, c],
                                          staging_register=k, mxu_index=m)
                    pltpu.matmul_acc_lhs(acc_addr=0, lhs=lhs_k[k],
                                         mxu_index=m, load_staged_rhs=k)
        for m, clist in seqs:
            if i < len(clist):
                cols[clist[i]] = pltpu.matmul_pop(
                    acc_addr=0, shape=(M, 256), dtype=jnp.float32,
                    mxu_index=m)
    return cols


def _gru_stack_kernel(x_ref, wih_ref, whh_ref, bi_ref, bhn_ref,
                      fcw_ref, fcb_ref, y_ref, *scratch):
    L = len(scratch) // 2
    seq_refs = scratch[:L]
    gi_refs = scratch[L:]
    T, Bb, H = seq_refs[0].shape
    C = gi_refs[0].shape[0]
    NC = T // C

    def gi_for(l, c):
        if l == 0:
            blk = x_ref[c * C:(c + 1) * C]
        else:
            blk = seq_refs[l - 1][c * C:(c + 1) * C]
        flat = blk.reshape(C * Bb, H).astype(jnp.bfloat16)
        cols = _explicit_gemm((flat[:, :256], flat[:, 256:]),
                              wih_ref[l], 6)
        gi = jnp.concatenate(cols, axis=1)
        gi_refs[l][...] = (gi + bi_ref[l]).reshape(C, Bb, 3 * H)

    def gates(l, gh, gi_t, h):
        r = 0.5 * jnp.tanh(0.5 * (gi_t[:, :H] + gh[:, :H])) + 0.5
        z = 0.5 * jnp.tanh(0.5 * (gi_t[:, H:2 * H] + gh[:, H:2 * H])) + 0.5
        n = jnp.tanh(gi_t[:, 2 * H:] + r * (gh[:, 2 * H:] + bhn_ref[l]))
        return n + z * (h - n)

    def multi_step(actives, d, s, hs):
        hb = {l: hs[i].astype(jnp.bfloat16) for i, l in enumerate(actives)}
        assign = {l: i % 2 for i, l in enumerate(actives)}
        base = {}
        nxt = [0, 0]
        for l in actives:
            base[l] = 24 * nxt[assign[l]]
            nxt[assign[l]] += 1
        for c in range(6):
            for k in range(2):
                for l in actives:
                    m = assign[l]
                    sr = (base[l] // 24 + k) % 2
                    pltpu.matmul_push_rhs(whh_ref[l, k, c],
                                          staging_register=sr, mxu_index=m)
                    pltpu.matmul_acc_lhs(
                        acc_addr=base[l] + 4 * c,
                        lhs=hb[l][:, 256 * k:256 * (k + 1)],
                        mxu_index=m, load_staged_rhs=sr)
        out = []
        for i, l in enumerate(actives):
            m = assign[l]
            cols = [pltpu.matmul_pop(acc_addr=base[l] + 4 * c,
                                     shape=(Bb, 256), dtype=jnp.float32,
                                     mxu_index=m)
                    for c in range(6)]
            gh = jnp.concatenate(cols, axis=1)
            h_new = gates(l, gh, gi_refs[l][s], hs[i])
            seq_refs[l][(d - l) * C + s] = h_new
            out.append(h_new)
        return tuple(out)

    hs = [jnp.zeros((Bb, H), jnp.float32) for _ in range(L)]
    for d in range(NC + L - 1):
        actives = [l for l in range(L) if 0 <= d - l < NC]
        for l in actives:
            gi_for(l, d - l)

        def body(s, carry, actives=actives, d=d):
            return multi_step(actives, d, s, carry)

        out = lax.fori_loop(0, C, body, tuple(hs[l] for l in actives))
        for l, h in zip(actives, out):
            hs[l] = h

    seq2 = seq_refs[L - 1][...].reshape(T * Bb, H).astype(jnp.bfloat16)
    ycols = _explicit_gemm((seq2[:, :256], seq2[:, 256:]), fcw_ref, 2)
    y = jnp.concatenate(ycols, axis=1)
    y_ref[...] = (y + fcb_ref[...]).reshape(T, Bb, -1)


def _run_stack(x_tbh, wih_tiles, whh_tiles, bi, bhn, fcw_tiles, fcb):
    T, B, H = x_tbh.shape
    L = wih_tiles.shape[0]
    O = fcb.shape[1]
    n_cores = 2 if (B % 16 == 0) else 1
    Bb = B // n_cores
    return pl.pallas_call(
        _gru_stack_kernel,
        out_shape=jax.ShapeDtypeStruct((T, B, O), jnp.float32),
        grid_spec=pltpu.PrefetchScalarGridSpec(
            num_scalar_prefetch=0,
            grid=(n_cores,),
            in_specs=[
                pl.BlockSpec((T, Bb, H), lambda i: (0, i, 0)),
                pl.BlockSpec((L, 2, 6, 256, 256),
                             lambda i: (0, 0, 0, 0, 0)),
                pl.BlockSpec((L, 2, 6, 256, 256),
                             lambda i: (0, 0, 0, 0, 0)),
                pl.BlockSpec((L, 1, 3 * H), lambda i: (0, 0, 0)),
                pl.BlockSpec((L, 1, H), lambda i: (0, 0, 0)),
                pl.BlockSpec((2, 2, 256, 256), lambda i: (0, 0, 0, 0)),
                pl.BlockSpec((1, O), lambda i: (0, 0)),
            ],
            out_specs=pl.BlockSpec((T, Bb, O), lambda i: (0, i, 0)),
            scratch_shapes=(
                [pltpu.VMEM((T, Bb, H), jnp.float32)] * L +
                [pltpu.VMEM((min(_CHUNK, T), Bb, 3 * H), jnp.float32)] * L),
        ),
        compiler_params=pltpu.CompilerParams(
            dimension_semantics=("parallel",)),
    )(x_tbh, wih_tiles, whh_tiles, bi, bhn, fcw_tiles, fcb)


def kernel(x,
           gru_w_ih_0, gru_w_hh_0, gru_b_ih_0, gru_b_hh_0,
           gru_w_ih_1, gru_w_hh_1, gru_b_ih_1, gru_b_hh_1,
           gru_w_ih_2, gru_w_hh_2, gru_b_ih_2, gru_b_hh_2,
           gru_w_ih_3, gru_w_hh_3, gru_b_ih_3, gru_b_hh_3,
           fc_w, fc_b):
    wihs = [gru_w_ih_0, gru_w_ih_1, gru_w_ih_2, gru_w_ih_3]
    whhs = [gru_w_hh_0, gru_w_hh_1, gru_w_hh_2, gru_w_hh_3]
    bihs = [gru_b_ih_0, gru_b_ih_1, gru_b_ih_2, gru_b_ih_3]
    bhhs = [gru_b_hh_0, gru_b_hh_1, gru_b_hh_2, gru_b_hh_3]

    B, H, T = x.shape
    x_tbh = jnp.transpose(x, (2, 0, 1)).astype(jnp.bfloat16)

    def tiled(w, ncols):
        return w.T.reshape(2, 256, ncols, 256).transpose(0, 2, 1, 3)

    wih_tiles = jnp.stack([tiled(w, 6)
                           for w in wihs]).astype(jnp.bfloat16)
    whh_tiles = jnp.stack([tiled(w, 6)
                           for w in whhs]).astype(jnp.bfloat16)
    bi = jnp.stack([
        (bih + jnp.concatenate([bhh[:2 * H], jnp.zeros((H,), bhh.dtype)]))
        .reshape(1, 3 * H)
        for bih, bhh in zip(bihs, bhhs)]).astype(jnp.float32)
    bhn = jnp.stack([bhh[2 * H:].reshape(1, H)
                     for bhh in bhhs]).astype(jnp.float32)

    fcw_tiles = tiled(fc_w, 2).astype(jnp.bfloat16)
    fcb = fc_b.reshape(1, -1).astype(jnp.float32)

    y_tbo = _run_stack(x_tbh, wih_tiles, whh_tiles, bi, bhn, fcw_tiles, fcb)
    return jnp.transpose(y_tbo, (1, 0, 2))


# speedup vs baseline: 1.5522x; 1.5522x over previous
import jax
import jax.numpy as jnp
from jax import lax
from jax.experimental import pallas as pl
from jax.experimental.pallas import tpu as pltpu

_CHUNK = 8


def _gru_stack_kernel(x_ref, wih_ref, whh_ref, bi_ref, bhn_ref,
                      fcw_ref, fcb_ref, y_ref, *scratch):
    L = len(scratch) // 2
    seq_refs = scratch[:L]
    gi_refs = scratch[L:]
    T, Bb, H = seq_refs[0].shape
    C = gi_refs[0].shape[0]
    NC = T // C

    def gi_for(l, c):
        if l == 0:
            blk = x_ref[c * C:(c + 1) * C]
        else:
            blk = seq_refs[l - 1][c * C:(c + 1) * C]
        flat = blk.reshape(C * Bb, H).astype(jnp.bfloat16)
        gi = jnp.dot(flat, wih_ref[l], preferred_element_type=jnp.float32)
        gi_refs[l][...] = (gi + bi_ref[l]).reshape(C, Bb, 3 * H)

    def step(l, c, s, h):
        gh = jnp.dot(h.astype(jnp.bfloat16), whh_ref[l],
                     preferred_element_type=jnp.float32)
        gi_t = gi_refs[l][s]
        r = 0.5 * jnp.tanh(0.5 * (gi_t[:, :H] + gh[:, :H])) + 0.5
        z = 0.5 * jnp.tanh(0.5 * (gi_t[:, H:2 * H] + gh[:, H:2 * H])) + 0.5
        n = jnp.tanh(gi_t[:, 2 * H:] + r * (gh[:, 2 * H:] + bhn_ref[l]))
        h_new = n + z * (h - n)
        seq_refs[l][c * C + s] = h_new
        return h_new

    hs = [jnp.zeros((Bb, H), jnp.float32) for _ in range(L)]
    for d in range(NC + L - 1):
        actives = [l for l in range(L) if 0 <= d - l < NC]
        for l in actives:
            gi_for(l, d - l)

        def body(j, carry, actives=actives, d=d):
            for u in range(2):
                carry = tuple(step(l, d - l, 2 * j + u, h)
                              for l, h in zip(actives, carry))
            return carry

        out = lax.fori_loop(0, C // 2, body, tuple(hs[l] for l in actives))
        for l, h in zip(actives, out):
            hs[l] = h

    seq2 = seq_refs[L - 1][...].reshape(T * Bb, H).astype(jnp.bfloat16)
    y = jnp.dot(seq2, fcw_ref[...], preferred_element_type=jnp.float32)
    y_ref[...] = (y + fcb_ref[...]).reshape(T, Bb, -1)


def _run_stack(x_tbh, wih, whh, bi, bhn, fcw, fcb):
    T, B, H = x_tbh.shape
    L = wih.shape[0]
    O = fcw.shape[1]
    n_cores = 2 if (B % 16 == 0) else 1
    Bb = B // n_cores
    return pl.pallas_call(
        _gru_stack_kernel,
        out_shape=jax.ShapeDtypeStruct((T, B, O), jnp.float32),
        grid_spec=pltpu.PrefetchScalarGridSpec(
            num_scalar_prefetch=0,
            grid=(n_cores,),
            in_specs=[
                pl.BlockSpec((T, Bb, H), lambda i: (0, i, 0)),
                pl.BlockSpec((L, H, 3 * H), lambda i: (0, 0, 0)),
                pl.BlockSpec((L, H, 3 * H), lambda i: (0, 0, 0)),
                pl.BlockSpec((L, 1, 3 * H), lambda i: (0, 0, 0)),
                pl.BlockSpec((L, 1, H), lambda i: (0, 0, 0)),
                pl.BlockSpec((H, O), lambda i: (0, 0)),
                pl.BlockSpec((1, O), lambda i: (0, 0)),
            ],
            out_specs=pl.BlockSpec((T, Bb, O), lambda i: (0, i, 0)),
            scratch_shapes=(
                [pltpu.VMEM((T, Bb, H), jnp.float32)] * L +
                [pltpu.VMEM((min(_CHUNK, T), Bb, 3 * H), jnp.float32)] * L),
        ),
        compiler_params=pltpu.CompilerParams(
            dimension_semantics=("parallel",)),
    )(x_tbh, wih, whh, bi, bhn, fcw, fcb)


def kernel(x,
           gru_w_ih_0, gru_w_hh_0, gru_b_ih_0, gru_b_hh_0,
           gru_w_ih_1, gru_w_hh_1, gru_b_ih_1, gru_b_hh_1,
           gru_w_ih_2, gru_w_hh_2, gru_b_ih_2, gru_b_hh_2,
           gru_w_ih_3, gru_w_hh_3, gru_b_ih_3, gru_b_hh_3,
           fc_w, fc_b):
    wihs = [gru_w_ih_0, gru_w_ih_1, gru_w_ih_2, gru_w_ih_3]
    whhs = [gru_w_hh_0, gru_w_hh_1, gru_w_hh_2, gru_w_hh_3]
    bihs = [gru_b_ih_0, gru_b_ih_1, gru_b_ih_2, gru_b_ih_3]
    bhhs = [gru_b_hh_0, gru_b_hh_1, gru_b_hh_2, gru_b_hh_3]

    B, H, T = x.shape
    x_tbh = jnp.transpose(x, (2, 0, 1)).astype(jnp.bfloat16)

    wih = jnp.stack([w.T for w in wihs]).astype(jnp.bfloat16)
    whh = jnp.stack([w.T for w in whhs]).astype(jnp.bfloat16)
    bi = jnp.stack([
        (bih + jnp.concatenate([bhh[:2 * H], jnp.zeros((H,), bhh.dtype)]))
        .reshape(1, 3 * H)
        for bih, bhh in zip(bihs, bhhs)]).astype(jnp.float32)
    bhn = jnp.stack([bhh[2 * H:].reshape(1, H)
                     for bhh in bhhs]).astype(jnp.float32)

    fcw = fc_w.T.astype(jnp.bfloat16)
    fcb = fc_b.reshape(1, -1).astype(jnp.float32)

    y_tbo = _run_stack(x_tbh, wih, whh, bi, bhn, fcw, fcb)
    return jnp.transpose(y_tbo, (1, 0, 2))
